# Initial kernel scaffold; baseline (speedup 1.0000x reference)
import jax
import jax.numpy as jnp
from jax.experimental import pallas as pl


def kernel(node_rep, relate_rep, relate_os, relate_mask, vision_feat, relation_mask, box_mask, node_mask, scale, W_map_v, W_edge, W_node, W_rel, W_fcv, b_fcv):
    raise NotImplementedError("write your pallas kernel here")


# speedup vs baseline: 1.3025x; 1.3025x over previous
import functools

import jax
import jax.numpy as jnp
from jax.experimental import pallas as pl

B, NODE, REL, NFEAT = 16, 6, 6, 64
DIM_V, DIM_WORD, DIM_VISION, DIM_EDGE, CLS_FC = 256, 512, 2048, 256, 1024

_F32 = jnp.float32


def _cmr_body(vf_ref, node_ref, rel_ref, subj_ref, obj_ref, relm_ref,
              boxm_ref, nodem_ref, relnm_ref, scale_ref, Wmap_ref, Wedge_ref,
              Wnode_ref, Wrel_ref, Wfcv_ref, bfcv_ref, mem_out, att_out):
    vf = vf_ref[0]
    scale = scale_ref[...]

    sq = jnp.sum(vf * vf, axis=0, keepdims=True)
    inv = 1.0 / jnp.sqrt(sq + 1e-12)

    vfs = vf * scale
    fmap0 = jax.lax.dot_general(vfs, Wmap_ref[...],
                                (((0,), (0,)), ((), ())),
                                preferred_element_type=_F32)

    We0 = Wedge_ref[0:DIM_V, :]
    We1 = Wedge_ref[DIM_V:2 * DIM_V, :]
    A0 = jnp.dot(fmap0, We0, preferred_element_type=_F32)
    C0 = jnp.dot(fmap0, We1, preferred_element_type=_F32)

    node_proj = jnp.dot(node_ref[0], Wnode_ref[...],
                        preferred_element_type=_F32)
    logits = jax.lax.dot_general(node_proj, fmap0,
                                 (((1,), (1,)), ((), ())),
                                 preferred_element_type=_F32) * inv
    boxm = boxm_ref[0]
    logits = jnp.where(boxm > 0.0, logits, -1e7)
    logits = logits - jnp.max(logits, axis=-1, keepdims=True)
    e = jnp.exp(logits)
    find = e / jnp.sum(e, axis=-1, keepdims=True)
    find = find * nodem_ref[0]

    rel_proj = jnp.dot(rel_ref[0], Wrel_ref[...],
                       preferred_element_type=_F32)
    P = jax.lax.dot_general(rel_proj, A0, (((1,), (1,)), ((), ())),
                            preferred_element_type=_F32) * inv
    Q = jax.lax.dot_general(rel_proj, C0, (((1,), (1,)), ((), ())),
                            preferred_element_type=_F32) * inv

    ks = jax.lax.broadcasted_iota(jnp.int32, (NODE, REL), 0)
    subj = subj_ref[0]
    obj = obj_ref[0]
    subj_oh = jnp.where((ks == jnp.clip(subj, 0, NODE - 1)) & (subj != -1),
                        1.0, 0.0).astype(_F32)
    obj_oh = jnp.where(ks == jnp.clip(obj, 0, NODE - 1),
                       1.0, 0.0).astype(_F32)

    g = jax.lax.dot_general(obj_oh, find, (((0,), (0,)), ((), ())),
                            preferred_element_type=_F32)

    w = jax.nn.sigmoid(Q[:, :, None] + P[:, None, :])
    w = w * relnm_ref[...]
    gathered = jnp.sum(g[:, :, None] * w, axis=1)
    gathered = gathered * relm_ref[0]

    find2 = find + jnp.dot(subj_oh, gathered, preferred_element_type=_F32)

    final_att = jnp.max(find2, axis=0, keepdims=True)
    norm = jnp.maximum(jnp.max(final_att), 1.0)
    final_att = final_att / norm
    final_att = final_att * boxm + (1.0 - boxm) * 1e-7

    att_out[0] = final_att

    mem = jax.lax.dot_general(final_att, vf, (((1,), (1,)), ((), ())),
                              preferred_element_type=_F32)
    mem_out[0] = jnp.dot(mem, Wfcv_ref[...],
                         preferred_element_type=_F32) + bfcv_ref[...]


def _run(node_rep, relate_rep, relate_os, relate_mask, vision_feat,
         relation_mask, box_mask, node_mask, scale, W_map_v, W_edge, W_node,
         W_rel, W_fcv, b_fcv, interpret=False):
    subj = relate_os[:, :, 1].reshape(B, 1, REL)
    obj = relate_os[:, :, 0].reshape(B, 1, REL)
    relm = relate_mask.reshape(B, REL, 1)
    boxm = box_mask.reshape(B, 1, NFEAT)
    nodem = node_mask.reshape(B, NODE, 1)
    scale2 = scale.reshape(DIM_VISION, 1)
    bfcv2 = b_fcv.reshape(1, CLS_FC)

    def const2(shape):
        return pl.BlockSpec(shape, lambda b: (0,) * len(shape))

    def batch3(shape):
        return pl.BlockSpec(shape, lambda b: (b, 0, 0))

    grid_spec = pl.GridSpec(
        grid=(B,),
        in_specs=[
            batch3((1, DIM_VISION, NFEAT)),
            batch3((1, NODE, DIM_WORD)),
            batch3((1, REL, DIM_WORD)),
            batch3((1, 1, REL)),
            batch3((1, 1, REL)),
            batch3((1, REL, 1)),
            batch3((1, 1, NFEAT)),
            batch3((1, NODE, 1)),
            batch3((1, NFEAT, NFEAT)),
            const2((DIM_VISION, 1)),
            const2((DIM_VISION, DIM_V)),
            const2((2 * DIM_V, DIM_EDGE)),
            const2((DIM_WORD, DIM_V)),
            const2((DIM_WORD, DIM_EDGE)),
            const2((DIM_VISION, CLS_FC)),
            const2((1, CLS_FC)),
        ],
        out_specs=[
            batch3((1, 1, CLS_FC)),
            batch3((1, 1, NFEAT)),
        ],
    )
    final_mem, final_att = pl.pallas_call(
        _cmr_body,
        grid_spec=grid_spec,
        out_shape=[
            jax.ShapeDtypeStruct((B, 1, CLS_FC), _F32),
            jax.ShapeDtypeStruct((B, 1, NFEAT), _F32),
        ],
        interpret=interpret,
    )(vision_feat, node_rep, relate_rep, subj, obj, relm, boxm, nodem,
      relation_mask, scale2, W_map_v, W_edge, W_node, W_rel, W_fcv, bfcv2)
    return final_mem.reshape(B, CLS_FC), final_att.reshape(B, NFEAT)


def kernel(node_rep, relate_rep, relate_os, relate_mask, vision_feat,
           relation_mask, box_mask, node_mask, scale, W_map_v, W_edge,
           W_node, W_rel, W_fcv, b_fcv):
    return _run(node_rep, relate_rep, relate_os, relate_mask, vision_feat,
                relation_mask, box_mask, node_mask, scale, W_map_v, W_edge,
                W_node, W_rel, W_fcv, b_fcv)


# speedup vs baseline: 1.4264x; 1.0950x over previous
import functools

import jax
import jax.numpy as jnp
from jax.experimental import pallas as pl

B, NODE, REL, NFEAT = 16, 6, 6, 64
DIM_V, DIM_WORD, DIM_VISION, DIM_EDGE, CLS_FC = 256, 512, 2048, 256, 1024

_F32 = jnp.float32


def _cmr_body(vf_ref, node_ref, rel_ref, subj_ref, obj_ref, relm_ref,
              boxm_ref, nodem_ref, relnm_ref, scale_ref, Wmap_ref, Wedge_ref,
              Wnode_ref, Wrel_ref, mem_out, att_out):
    vf = vf_ref[0]
    scale = scale_ref[...]

    sq = jnp.sum(vf * vf, axis=0, keepdims=True)
    inv = 1.0 / jnp.sqrt(sq + 1e-12)

    vfs = vf * scale
    fmap0 = jax.lax.dot_general(vfs, Wmap_ref[...],
                                (((0,), (0,)), ((), ())),
                                preferred_element_type=_F32)

    We0 = Wedge_ref[0:DIM_V, :]
    We1 = Wedge_ref[DIM_V:2 * DIM_V, :]
    A0 = jnp.dot(fmap0, We0, preferred_element_type=_F32)
    C0 = jnp.dot(fmap0, We1, preferred_element_type=_F32)

    node_proj = jnp.dot(node_ref[0], Wnode_ref[...],
                        preferred_element_type=_F32)
    logits = jax.lax.dot_general(node_proj, fmap0,
                                 (((1,), (1,)), ((), ())),
                                 preferred_element_type=_F32) * inv
    boxm = boxm_ref[0]
    logits = jnp.where(boxm > 0.0, logits, -1e7)
    logits = logits - jnp.max(logits, axis=-1, keepdims=True)
    e = jnp.exp(logits)
    find = e / jnp.sum(e, axis=-1, keepdims=True)
    find = find * nodem_ref[0]

    rel_proj = jnp.dot(rel_ref[0], Wrel_ref[...],
                       preferred_element_type=_F32)
    P = jax.lax.dot_general(rel_proj, A0, (((1,), (1,)), ((), ())),
                            preferred_element_type=_F32) * inv
    Q = jax.lax.dot_general(rel_proj, C0, (((1,), (1,)), ((), ())),
                            preferred_element_type=_F32) * inv

    ks = jax.lax.broadcasted_iota(jnp.int32, (NODE, REL), 0)
    subj = subj_ref[0]
    obj = obj_ref[0]
    subj_oh = jnp.where((ks == jnp.clip(subj, 0, NODE - 1)) & (subj != -1),
                        1.0, 0.0).astype(_F32)
    obj_oh = jnp.where(ks == jnp.clip(obj, 0, NODE - 1),
                       1.0, 0.0).astype(_F32)

    g = jax.lax.dot_general(obj_oh, find, (((0,), (0,)), ((), ())),
                            preferred_element_type=_F32)

    w = jax.nn.sigmoid(Q[:, :, None] + P[:, None, :])
    w = w * relnm_ref[...]
    gathered = jnp.sum(g[:, :, None] * w, axis=1)
    gathered = gathered * relm_ref[0]

    find2 = find + jnp.dot(subj_oh, gathered, preferred_element_type=_F32)

    final_att = jnp.max(find2, axis=0, keepdims=True)
    norm = jnp.maximum(jnp.max(final_att), 1.0)
    final_att = final_att / norm
    final_att = final_att * boxm + (1.0 - boxm) * 1e-7

    att_out[0] = final_att

    mem_out[0] = jax.lax.dot_general(final_att, vf, (((1,), (1,)), ((), ())),
                                     preferred_element_type=_F32)


def _fc_body(mem_ref, Wfcv_ref, bfcv_ref, out_ref):
    out_ref[...] = jnp.dot(mem_ref[...], Wfcv_ref[...],
                           preferred_element_type=_F32) + bfcv_ref[...]


def _run(node_rep, relate_rep, relate_os, relate_mask, vision_feat,
         relation_mask, box_mask, node_mask, scale, W_map_v, W_edge, W_node,
         W_rel, W_fcv, b_fcv, interpret=False):
    subj = relate_os[:, :, 1].reshape(B, 1, REL)
    obj = relate_os[:, :, 0].reshape(B, 1, REL)
    relm = relate_mask.reshape(B, REL, 1)
    boxm = box_mask.reshape(B, 1, NFEAT)
    nodem = node_mask.reshape(B, NODE, 1)
    scale2 = scale.reshape(DIM_VISION, 1)
    bfcv2 = b_fcv.reshape(1, CLS_FC)

    def const2(shape):
        return pl.BlockSpec(shape, lambda b: (0,) * len(shape))

    def batch3(shape):
        return pl.BlockSpec(shape, lambda b: (b, 0, 0))

    grid_spec = pl.GridSpec(
        grid=(B,),
        in_specs=[
            batch3((1, DIM_VISION, NFEAT)),
            batch3((1, NODE, DIM_WORD)),
            batch3((1, REL, DIM_WORD)),
            batch3((1, 1, REL)),
            batch3((1, 1, REL)),
            batch3((1, REL, 1)),
            batch3((1, 1, NFEAT)),
            batch3((1, NODE, 1)),
            batch3((1, NFEAT, NFEAT)),
            const2((DIM_VISION, 1)),
            const2((DIM_VISION, DIM_V)),
            const2((2 * DIM_V, DIM_EDGE)),
            const2((DIM_WORD, DIM_V)),
            const2((DIM_WORD, DIM_EDGE)),
        ],
        out_specs=[
            batch3((1, 1, DIM_VISION)),
            batch3((1, 1, NFEAT)),
        ],
    )
    mem, final_att = pl.pallas_call(
        _cmr_body,
        grid_spec=grid_spec,
        out_shape=[
            jax.ShapeDtypeStruct((B, 1, DIM_VISION), _F32),
            jax.ShapeDtypeStruct((B, 1, NFEAT), _F32),
        ],
        interpret=interpret,
    )(vision_feat, node_rep, relate_rep, subj, obj, relm, boxm, nodem,
      relation_mask, scale2, W_map_v, W_edge, W_node, W_rel)

    final_mem = pl.pallas_call(
        _fc_body,
        out_shape=jax.ShapeDtypeStruct((B, CLS_FC), _F32),
        interpret=interpret,
    )(mem.reshape(B, DIM_VISION), W_fcv, bfcv2)
    return final_mem, final_att.reshape(B, NFEAT)


def kernel(node_rep, relate_rep, relate_os, relate_mask, vision_feat,
           relation_mask, box_mask, node_mask, scale, W_map_v, W_edge,
           W_node, W_rel, W_fcv, b_fcv):
    return _run(node_rep, relate_rep, relate_os, relate_mask, vision_feat,
                relation_mask, box_mask, node_mask, scale, W_map_v, W_edge,
                W_node, W_rel, W_fcv, b_fcv)


# speedup vs baseline: 1.4861x; 1.0419x over previous
import functools

import jax
import jax.numpy as jnp
from jax.experimental import pallas as pl

B, NODE, REL, NFEAT = 16, 6, 6, 64
DIM_V, DIM_WORD, DIM_VISION, DIM_EDGE, CLS_FC = 256, 512, 2048, 256, 1024

_F32 = jnp.float32


UNROLL = 2


def _cmr_body(vf_ref, node_ref, rel_ref, subj_ref, obj_ref, relm_ref,
              boxm_ref, nodem_ref, relnm_ref, scale_ref, Wmap_ref, Wedge_ref,
              Wnode_ref, Wrel_ref, mem_out, att_out):
    for i in range(UNROLL):
        _cmr_one(i, vf_ref, node_ref, rel_ref, subj_ref, obj_ref, relm_ref,
                 boxm_ref, nodem_ref, relnm_ref, scale_ref, Wmap_ref,
                 Wedge_ref, Wnode_ref, Wrel_ref, mem_out, att_out)


def _cmr_one(i, vf_ref, node_ref, rel_ref, subj_ref, obj_ref, relm_ref,
             boxm_ref, nodem_ref, relnm_ref, scale_ref, Wmap_ref, Wedge_ref,
             Wnode_ref, Wrel_ref, mem_out, att_out):
    vf = vf_ref[i]
    scale = scale_ref[...]

    sq = jnp.sum(vf * vf, axis=0, keepdims=True)
    inv = 1.0 / jnp.sqrt(sq + 1e-12)

    vfs = vf * scale
    fmap0 = jax.lax.dot_general(vfs, Wmap_ref[...],
                                (((0,), (0,)), ((), ())),
                                preferred_element_type=_F32)

    We0 = Wedge_ref[0:DIM_V, :]
    We1 = Wedge_ref[DIM_V:2 * DIM_V, :]
    A0 = jnp.dot(fmap0, We0, preferred_element_type=_F32)
    C0 = jnp.dot(fmap0, We1, preferred_element_type=_F32)

    node_proj = jnp.dot(node_ref[i], Wnode_ref[...],
                        preferred_element_type=_F32)
    logits = jax.lax.dot_general(node_proj, fmap0,
                                 (((1,), (1,)), ((), ())),
                                 preferred_element_type=_F32) * inv
    boxm = boxm_ref[i]
    logits = jnp.where(boxm > 0.0, logits, -1e7)
    logits = logits - jnp.max(logits, axis=-1, keepdims=True)
    e = jnp.exp(logits)
    find = e / jnp.sum(e, axis=-1, keepdims=True)
    find = find * nodem_ref[i]

    rel_proj = jnp.dot(rel_ref[i], Wrel_ref[...],
                       preferred_element_type=_F32)
    P = jax.lax.dot_general(rel_proj, A0, (((1,), (1,)), ((), ())),
                            preferred_element_type=_F32) * inv
    Q = jax.lax.dot_general(rel_proj, C0, (((1,), (1,)), ((), ())),
                            preferred_element_type=_F32) * inv

    ks = jax.lax.broadcasted_iota(jnp.int32, (NODE, REL), 0)
    subj = subj_ref[i]
    obj = obj_ref[i]
    subj_oh = jnp.where((ks == jnp.clip(subj, 0, NODE - 1)) & (subj != -1),
                        1.0, 0.0).astype(_F32)
    obj_oh = jnp.where(ks == jnp.clip(obj, 0, NODE - 1),
                       1.0, 0.0).astype(_F32)

    g = jax.lax.dot_general(obj_oh, find, (((0,), (0,)), ((), ())),
                            preferred_element_type=_F32)

    w = jax.nn.sigmoid(Q[:, :, None] + P[:, None, :])
    w = w * relnm_ref[i]
    gathered = jnp.sum(g[:, :, None] * w, axis=1)
    gathered = gathered * relm_ref[i]

    find2 = find + jnp.dot(subj_oh, gathered, preferred_element_type=_F32)

    final_att = jnp.max(find2, axis=0, keepdims=True)
    norm = jnp.maximum(jnp.max(final_att), 1.0)
    final_att = final_att / norm
    final_att = final_att * boxm + (1.0 - boxm) * 1e-7

    att_out[i] = final_att

    mem_out[i] = jax.lax.dot_general(final_att, vf, (((1,), (1,)), ((), ())),
                                     preferred_element_type=_F32)


def _fc_body(mem_ref, Wfcv_ref, bfcv_ref, out_ref):
    out_ref[...] = jnp.dot(mem_ref[...], Wfcv_ref[...],
                           preferred_element_type=_F32) + bfcv_ref[...]


def _run(node_rep, relate_rep, relate_os, relate_mask, vision_feat,
         relation_mask, box_mask, node_mask, scale, W_map_v, W_edge, W_node,
         W_rel, W_fcv, b_fcv, interpret=False):
    subj = relate_os[:, :, 1].reshape(B, 1, REL)
    obj = relate_os[:, :, 0].reshape(B, 1, REL)
    relm = relate_mask.reshape(B, REL, 1)
    boxm = box_mask.reshape(B, 1, NFEAT)
    nodem = node_mask.reshape(B, NODE, 1)
    scale2 = scale.reshape(DIM_VISION, 1)
    bfcv2 = b_fcv.reshape(1, CLS_FC)

    def const2(shape):
        return pl.BlockSpec(shape, lambda b: (0,) * len(shape))

    def batch3(shape):
        return pl.BlockSpec(shape, lambda b: (b, 0, 0))

    U = UNROLL
    grid_spec = pl.GridSpec(
        grid=(B // U,),
        in_specs=[
            batch3((U, DIM_VISION, NFEAT)),
            batch3((U, NODE, DIM_WORD)),
            batch3((U, REL, DIM_WORD)),
            batch3((U, 1, REL)),
            batch3((U, 1, REL)),
            batch3((U, REL, 1)),
            batch3((U, 1, NFEAT)),
            batch3((U, NODE, 1)),
            batch3((U, NFEAT, NFEAT)),
            const2((DIM_VISION, 1)),
            const2((DIM_VISION, DIM_V)),
            const2((2 * DIM_V, DIM_EDGE)),
            const2((DIM_WORD, DIM_V)),
            const2((DIM_WORD, DIM_EDGE)),
        ],
        out_specs=[
            batch3((U, 1, DIM_VISION)),
            batch3((U, 1, NFEAT)),
        ],
    )
    mem, final_att = pl.pallas_call(
        _cmr_body,
        grid_spec=grid_spec,
        out_shape=[
            jax.ShapeDtypeStruct((B, 1, DIM_VISION), _F32),
            jax.ShapeDtypeStruct((B, 1, NFEAT), _F32),
        ],
        interpret=interpret,
    )(vision_feat, node_rep, relate_rep, subj, obj, relm, boxm, nodem,
      relation_mask, scale2, W_map_v, W_edge, W_node, W_rel)

    final_mem = pl.pallas_call(
        _fc_body,
        out_shape=jax.ShapeDtypeStruct((B, CLS_FC), _F32),
        interpret=interpret,
    )(mem.reshape(B, DIM_VISION), W_fcv, bfcv2)
    return final_mem, final_att.reshape(B, NFEAT)


def kernel(node_rep, relate_rep, relate_os, relate_mask, vision_feat,
           relation_mask, box_mask, node_mask, scale, W_map_v, W_edge,
           W_node, W_rel, W_fcv, b_fcv):
    return _run(node_rep, relate_rep, relate_os, relate_mask, vision_feat,
                relation_mask, box_mask, node_mask, scale, W_map_v, W_edge,
                W_node, W_rel, W_fcv, b_fcv)


# speedup vs baseline: 1.5151x; 1.0195x over previous
import functools

import jax
import jax.numpy as jnp
from jax.experimental import pallas as pl

B, NODE, REL, NFEAT = 16, 6, 6, 64
DIM_V, DIM_WORD, DIM_VISION, DIM_EDGE, CLS_FC = 256, 512, 2048, 256, 1024

_F32 = jnp.float32


UNROLL = 4


def _cmr_body(vf_ref, node_ref, rel_ref, subj_ref, obj_ref, relm_ref,
              boxm_ref, nodem_ref, relnm_ref, scale_ref, Wmap_ref, Wedge_ref,
              Wnode_ref, Wrel_ref, mem_out, att_out):
    for i in range(UNROLL):
        _cmr_one(i, vf_ref, node_ref, rel_ref, subj_ref, obj_ref, relm_ref,
                 boxm_ref, nodem_ref, relnm_ref, scale_ref, Wmap_ref,
                 Wedge_ref, Wnode_ref, Wrel_ref, mem_out, att_out)


def _cmr_one(i, vf_ref, node_ref, rel_ref, subj_ref, obj_ref, relm_ref,
             boxm_ref, nodem_ref, relnm_ref, scale_ref, Wmap_ref, Wedge_ref,
             Wnode_ref, Wrel_ref, mem_out, att_out):
    vf = vf_ref[i]
    scale = scale_ref[...]

    sq = jnp.sum(vf * vf, axis=0, keepdims=True)
    inv = 1.0 / jnp.sqrt(sq + 1e-12)

    vfs = vf * scale
    fmap0 = jax.lax.dot_general(vfs, Wmap_ref[...],
                                (((0,), (0,)), ((), ())),
                                preferred_element_type=_F32)

    We0 = Wedge_ref[0:DIM_V, :]
    We1 = Wedge_ref[DIM_V:2 * DIM_V, :]
    A0 = jnp.dot(fmap0, We0, preferred_element_type=_F32)
    C0 = jnp.dot(fmap0, We1, preferred_element_type=_F32)

    node_proj = jnp.dot(node_ref[i], Wnode_ref[...],
                        preferred_element_type=_F32)
    logits = jax.lax.dot_general(node_proj, fmap0,
                                 (((1,), (1,)), ((), ())),
                                 preferred_element_type=_F32) * inv
    boxm = boxm_ref[i]
    logits = jnp.where(boxm > 0.0, logits, -1e7)
    logits = logits - jnp.max(logits, axis=-1, keepdims=True)
    e = jnp.exp(logits)
    find = e / jnp.sum(e, axis=-1, keepdims=True)
    find = find * nodem_ref[i]

    rel_proj = jnp.dot(rel_ref[i], Wrel_ref[...],
                       preferred_element_type=_F32)
    P = jax.lax.dot_general(rel_proj, A0, (((1,), (1,)), ((), ())),
                            preferred_element_type=_F32) * inv
    Q = jax.lax.dot_general(rel_proj, C0, (((1,), (1,)), ((), ())),
                            preferred_element_type=_F32) * inv

    ks = jax.lax.broadcasted_iota(jnp.int32, (NODE, REL), 0)
    subj = subj_ref[i]
    obj = obj_ref[i]
    subj_oh = jnp.where((ks == jnp.clip(subj, 0, NODE - 1)) & (subj != -1),
                        1.0, 0.0).astype(_F32)
    obj_oh = jnp.where(ks == jnp.clip(obj, 0, NODE - 1),
                       1.0, 0.0).astype(_F32)

    g = jax.lax.dot_general(obj_oh, find, (((0,), (0,)), ((), ())),
                            preferred_element_type=_F32)

    w = jax.nn.sigmoid(Q[:, :, None] + P[:, None, :])
    w = w * relnm_ref[i]
    gathered = jnp.sum(g[:, :, None] * w, axis=1)
    gathered = gathered * relm_ref[i]

    find2 = find + jnp.dot(subj_oh, gathered, preferred_element_type=_F32)

    final_att = jnp.max(find2, axis=0, keepdims=True)
    norm = jnp.maximum(jnp.max(final_att), 1.0)
    final_att = final_att / norm
    final_att = final_att * boxm + (1.0 - boxm) * 1e-7

    att_out[i] = final_att

    mem_out[i] = jax.lax.dot_general(final_att, vf, (((1,), (1,)), ((), ())),
                                     preferred_element_type=_F32)


def _fc_body(mem_ref, Wfcv_ref, bfcv_ref, out_ref):
    out_ref[...] = jnp.dot(mem_ref[...], Wfcv_ref[...],
                           preferred_element_type=_F32) + bfcv_ref[...]


def _run(node_rep, relate_rep, relate_os, relate_mask, vision_feat,
         relation_mask, box_mask, node_mask, scale, W_map_v, W_edge, W_node,
         W_rel, W_fcv, b_fcv, interpret=False):
    subj = relate_os[:, :, 1].reshape(B, 1, REL)
    obj = relate_os[:, :, 0].reshape(B, 1, REL)
    relm = relate_mask.reshape(B, REL, 1)
    boxm = box_mask.reshape(B, 1, NFEAT)
    nodem = node_mask.reshape(B, NODE, 1)
    scale2 = scale.reshape(DIM_VISION, 1)
    bfcv2 = b_fcv.reshape(1, CLS_FC)

    def const2(shape):
        return pl.BlockSpec(shape, lambda b: (0,) * len(shape))

    def batch3(shape):
        return pl.BlockSpec(shape, lambda b: (b, 0, 0))

    U = UNROLL
    grid_spec = pl.GridSpec(
        grid=(B // U,),
        in_specs=[
            batch3((U, DIM_VISION, NFEAT)),
            batch3((U, NODE, DIM_WORD)),
            batch3((U, REL, DIM_WORD)),
            batch3((U, 1, REL)),
            batch3((U, 1, REL)),
            batch3((U, REL, 1)),
            batch3((U, 1, NFEAT)),
            batch3((U, NODE, 1)),
            batch3((U, NFEAT, NFEAT)),
            const2((DIM_VISION, 1)),
            const2((DIM_VISION, DIM_V)),
            const2((2 * DIM_V, DIM_EDGE)),
            const2((DIM_WORD, DIM_V)),
            const2((DIM_WORD, DIM_EDGE)),
        ],
        out_specs=[
            batch3((U, 1, DIM_VISION)),
            batch3((U, 1, NFEAT)),
        ],
    )
    mem, final_att = pl.pallas_call(
        _cmr_body,
        grid_spec=grid_spec,
        out_shape=[
            jax.ShapeDtypeStruct((B, 1, DIM_VISION), _F32),
            jax.ShapeDtypeStruct((B, 1, NFEAT), _F32),
        ],
        interpret=interpret,
    )(vision_feat, node_rep, relate_rep, subj, obj, relm, boxm, nodem,
      relation_mask, scale2, W_map_v, W_edge, W_node, W_rel)

    final_mem = pl.pallas_call(
        _fc_body,
        out_shape=jax.ShapeDtypeStruct((B, CLS_FC), _F32),
        interpret=interpret,
    )(mem.reshape(B, DIM_VISION), W_fcv, bfcv2)
    return final_mem, final_att.reshape(B, NFEAT)


def kernel(node_rep, relate_rep, relate_os, relate_mask, vision_feat,
           relation_mask, box_mask, node_mask, scale, W_map_v, W_edge,
           W_node, W_rel, W_fcv, b_fcv):
    return _run(node_rep, relate_rep, relate_os, relate_mask, vision_feat,
                relation_mask, box_mask, node_mask, scale, W_map_v, W_edge,
                W_node, W_rel, W_fcv, b_fcv)
